# baseline (speedup 1.0000x reference)
import functools

import jax
import jax.numpy as jnp
from jax import lax
from jax.experimental import pallas as pl
from jax.experimental.pallas import tpu as pltpu
from jax.experimental.pallas import tpu_sc as plsc

B, A, H, W = 32, 9, 37, 62
RP = B * A * H * 2
RR = B * A * H * 4
JOB = 296
NJP = RP // JOB
NJR = RR // JOB
COLS = (0, 16, 32, 46)

_mesh = plsc.VectorSubcoreMesh(core_axis_name="c", subcore_axis_name="s")


@functools.partial(
    pl.kernel,
    out_type=[
        jax.ShapeDtypeStruct((RP, W), jnp.float32),
        jax.ShapeDtypeStruct((RR, W), jnp.float32),
    ],
    mesh=_mesh,
    scratch_types=[
        pltpu.VMEM((8, H, W), jnp.float32),
        pltpu.VMEM((JOB, W), jnp.float32),
        pltpu.SemaphoreType.DMA,
    ],
)
def _sc_interleave(preds_hbm, regs_hbm, outp_hbm, outr_hbm, planes, obuf, sem):
    w = lax.axis_index("s") * 2 + lax.axis_index("c")

    def do_job(jid, src, out, nunit, k):
        cps = []
        for q in range(nunit):
            u = jid * nunit + q
            b = u // A
            a = u - b * A
            for c in range(k):
                cps.append(pltpu.async_copy(
                    src.at[b, c * A + a], planes.at[q * k + c], sem))
        for cp in cps:
            cp.wait()

        def body(h, _):
            for q in range(nunit):
                for c in range(k):
                    row = q * (H * k) + h * k + c
                    for col in COLS:
                        obuf[row, pl.ds(col, 16)] = (
                            planes[q * k + c, h, pl.ds(col, 16)])
            return 0

        lax.fori_loop(0, H, body, 0)
        pltpu.sync_copy(obuf, out.at[pl.ds(jid * JOB, JOB)])

    for t in range(3):
        jid = w + 32 * t
        @pl.when(jid < NJP)
        def _():
            do_job(jid, preds_hbm, outp_hbm, 4, 2)
    for t in range(5):
        jid = w + 32 * t
        @pl.when(jid < NJR)
        def _():
            do_job(jid, regs_hbm, outr_hbm, 2, 4)


def kernel(preds, regs):
    bs, _, fh, fw = preds.shape
    outp, outr = _sc_interleave(preds, regs)
    return (
        outp.reshape(bs, A, fh, 2, fw).transpose(0, 1, 2, 4, 3),
        outr.reshape(bs, A, fh, 4, fw).transpose(0, 1, 2, 4, 3),
    )


# speedup vs baseline: 1.2657x; 1.2657x over previous
import functools

import jax
import jax.numpy as jnp
from jax import lax
from jax.experimental import pallas as pl
from jax.experimental.pallas import tpu as pltpu
from jax.experimental.pallas import tpu_sc as plsc

B, A, H, W = 32, 9, 37, 62
RP = B * A * H * 2
RR = B * A * H * 4
JOB = 296
STG = 148
NJP = RP // JOB
NJR = RR // JOB
COLS = (0, 16, 32, 46)

_mesh = plsc.VectorSubcoreMesh(core_axis_name="c", subcore_axis_name="s")


@functools.partial(
    pl.kernel,
    out_type=[
        jax.ShapeDtypeStruct((RP, W), jnp.float32),
        jax.ShapeDtypeStruct((RR, W), jnp.float32),
    ],
    mesh=_mesh,
    scratch_types=[
        pltpu.VMEM((4, H, W), jnp.float32),
        pltpu.VMEM((4, H, W), jnp.float32),
        pltpu.VMEM((JOB, W), jnp.float32),
        pltpu.VMEM((JOB, W), jnp.float32),
        pltpu.SemaphoreType.DMA,
        pltpu.SemaphoreType.DMA,
        pltpu.SemaphoreType.DMA,
        pltpu.SemaphoreType.DMA,
    ],
)
def _sc_interleave(preds_hbm, regs_hbm, outp_hbm, outr_hbm,
                   pbuf0, pbuf1, obuf0, obuf1, sg0, sg1, so0, so1):
    w = lax.axis_index("s") * 2 + lax.axis_index("c")
    pbufs, obufs = (pbuf0, pbuf1), (obuf0, obuf1)
    sgs, sos = (sg0, sg1), (so0, so1)

    stages = [("p", s, st) for s in range(3) for st in (0, 1)]
    stages += [("r", s, st) for s in range(5) for st in (0, 1)]

    def stage_info(i):
        kind, slot, st = stages[i]
        jid = w + 32 * slot
        njobs = NJP if kind == "p" else NJR
        return kind, jid, st, jid < njobs

    def make_gathers(i):
        kind, jid, st, valid = stage_info(i)
        pbuf, sg = pbufs[i % 2], sgs[i % 2]
        cps = []
        for k in range(4):
            if kind == "p":
                u = 4 * jid + 2 * st + k // 2
                ch = (k % 2) * A + (u - (u // A) * A)
                src = preds_hbm
            else:
                u = 2 * jid + st
                ch = k * A + (u - (u // A) * A)
                src = regs_hbm
            cps.append(pltpu.make_async_copy(
                src.at[u // A, ch], pbuf.at[k], sg))
        return cps, valid

    def fire_gathers(g):
        cps, valid = g

        @pl.when(valid)
        def _():
            for cp in cps:
                cp.start()

    def wait_gathers(g):
        cps, valid = g

        @pl.when(valid)
        def _():
            for cp in cps:
                cp.wait()

    def interleave(i):
        kind, jid, st, valid = stage_info(i)
        pbuf = pbufs[i % 2]
        obuf = obufs[(i // 2) % 2]
        base = st * STG

        @pl.when(valid)
        def _():
            def body(h, _):
                for k in range(4):
                    if kind == "p":
                        row = base + (k // 2) * 74 + h * 2 + (k % 2)
                    else:
                        row = base + h * 4 + k
                    for col in COLS:
                        obuf[row, pl.ds(col, 16)] = pbuf[k, h, pl.ds(col, 16)]
                return 0
            lax.fori_loop(0, H, body, 0)

    def make_out(i):
        kind, jid, st, valid = stage_info(i)
        j = i // 2
        obuf, so = obufs[j % 2], sos[j % 2]
        out = outp_hbm if kind == "p" else outr_hbm
        return pltpu.make_async_copy(
            obuf, out.at[pl.ds(jid * JOB, JOB)], so), valid

    def start_out(o):
        cp, valid = o

        @pl.when(valid)
        def _():
            cp.start()

    def wait_out(o):
        cp, valid = o

        @pl.when(valid)
        def _():
            cp.wait()

    n = len(stages)
    pending_out = {}
    g = make_gathers(0)
    fire_gathers(g)
    for i in range(n):
        g_next = make_gathers(i + 1) if i + 1 < n else None
        if g_next is not None:
            fire_gathers(g_next)
        wait_gathers(g)
        g = g_next
        j = i // 2
        if stages[i][2] == 0 and (j - 2) in pending_out:
            wait_out(pending_out.pop(j - 2))
        interleave(i)
        if stages[i][2] == 1:
            o = make_out(i)
            start_out(o)
            pending_out[j] = o
    for o in pending_out.values():
        wait_out(o)


def kernel(preds, regs):
    bs, _, fh, fw = preds.shape
    outp, outr = _sc_interleave(preds, regs)
    return (
        outp.reshape(bs, A, fh, 2, fw).transpose(0, 1, 2, 4, 3),
        outr.reshape(bs, A, fh, 4, fw).transpose(0, 1, 2, 4, 3),
    )


# speedup vs baseline: 1.8282x; 1.4445x over previous
import functools

import jax
import jax.numpy as jnp
from jax import lax
from jax.experimental import pallas as pl
from jax.experimental.pallas import tpu as pltpu
from jax.experimental.pallas import tpu_sc as plsc

B, A, H, W = 32, 9, 37, 62
RP = B * A * H * 2
RR = B * A * H * 4
JOB = 296
STG = 148
NJR = RR // JOB
COLS = (0, 16, 32, 46)

_mesh = plsc.VectorSubcoreMesh(core_axis_name="c", subcore_axis_name="s")


@functools.partial(
    pl.kernel,
    out_type=jax.ShapeDtypeStruct((RR, W), jnp.float32),
    mesh=_mesh,
    scratch_types=[
        pltpu.VMEM((4, H, W), jnp.float32),
        pltpu.VMEM((4, H, W), jnp.float32),
        pltpu.VMEM((JOB, W), jnp.float32),
        pltpu.VMEM((JOB, W), jnp.float32),
        pltpu.SemaphoreType.DMA,
        pltpu.SemaphoreType.DMA,
        pltpu.SemaphoreType.DMA,
        pltpu.SemaphoreType.DMA,
    ],
)
def _sc_regs(regs_hbm, outr_hbm, pbuf0, pbuf1, obuf0, obuf1,
             sg0, sg1, so0, so1):
    w = lax.axis_index("s") * 2 + lax.axis_index("c")
    pbufs, obufs = (pbuf0, pbuf1), (obuf0, obuf1)
    sgs, sos = (sg0, sg1), (so0, so1)

    nstage = 10

    def stage_info(i):
        slot, st = i // 2, i % 2
        jid = w + 32 * slot
        return jid, st, jid < NJR

    def make_gathers(i):
        jid, st, valid = stage_info(i)
        pbuf, sg = pbufs[i % 2], sgs[i % 2]
        u = 2 * jid + st
        b = u // A
        a = u - b * A
        cps = [pltpu.make_async_copy(regs_hbm.at[b, k * A + a],
                                     pbuf.at[k], sg) for k in range(4)]
        return cps, valid

    def guarded(fn, valid):
        @pl.when(valid)
        def _():
            fn()

    def interleave(i):
        jid, st, valid = stage_info(i)
        pbuf = pbufs[i % 2]
        obuf = obufs[(i // 2) % 2]
        base = st * STG

        @pl.when(valid)
        def _():
            def body(h, _):
                for k in range(4):
                    row = base + h * 4 + k
                    for col in COLS:
                        obuf[row, pl.ds(col, 16)] = pbuf[k, h, pl.ds(col, 16)]
                return 0
            lax.fori_loop(0, H, body, 0)

    def make_out(i):
        jid, st, valid = stage_info(i)
        j = i // 2
        return pltpu.make_async_copy(
            obufs[j % 2], outr_hbm.at[pl.ds(jid * JOB, JOB)],
            sos[j % 2]), valid

    pending = {}
    g = make_gathers(0)
    guarded(lambda cps=g[0]: [c.start() for c in cps], g[1])
    for i in range(nstage):
        if i + 1 < nstage:
            gn = make_gathers(i + 1)
            guarded(lambda cps=gn[0]: [c.start() for c in cps], gn[1])
        else:
            gn = None
        guarded(lambda cps=g[0]: [c.wait() for c in cps], g[1])
        g = gn
        j = i // 2
        if i % 2 == 0 and (j - 2) in pending:
            cp, v = pending.pop(j - 2)
            guarded(lambda cp=cp: cp.wait(), v)
        interleave(i)
        if i % 2 == 1:
            cp, v = make_out(i)
            guarded(lambda cp=cp: cp.start(), v)
            pending[j] = (cp, v)
    for cp, v in pending.values():
        guarded(lambda cp=cp: cp.wait(), v)


OCT = B // 8
OROWS = RP // OCT


def _tc_preds_body(x_ref, o_ref):
    x = x_ref[...]
    y = x.reshape(2, A, H, 8, W).transpose(3, 1, 2, 0, 4)
    o_ref[...] = y.reshape(OROWS, W)


def _tc_preds(pt4):
    return pl.pallas_call(
        _tc_preds_body,
        out_shape=jax.ShapeDtypeStruct((RP, W), jnp.float32),
        grid=(OCT,),
        in_specs=[pl.BlockSpec((2 * A, H, 8, W), lambda q: (0, 0, q, 0))],
        out_specs=pl.BlockSpec((OROWS, W), lambda q: (q, 0)),
    )(pt4)


def kernel(preds, regs):
    bs, _, fh, fw = preds.shape
    pt4 = preds.transpose(1, 2, 0, 3)
    outp = _tc_preds(pt4)
    outr = _sc_regs(regs)
    return (
        outp.reshape(bs, A, fh, 2, fw).transpose(0, 1, 2, 4, 3),
        outr.reshape(bs, A, fh, 4, fw).transpose(0, 1, 2, 4, 3),
    )


# speedup vs baseline: 2.1550x; 1.1787x over previous
import functools

import jax
import jax.numpy as jnp
from jax import lax
from jax.experimental import pallas as pl
from jax.experimental.pallas import tpu as pltpu
from jax.experimental.pallas import tpu_sc as plsc

B, A, H, W = 32, 9, 37, 62
RP = B * A * H * 2
RR = B * A * H * 4
JOB = 296
STG = 148
NJP = RP // JOB
COLS = (0, 16, 32, 46)

_mesh = plsc.VectorSubcoreMesh(core_axis_name="c", subcore_axis_name="s")


@functools.partial(
    pl.kernel,
    out_type=jax.ShapeDtypeStruct((RP, W), jnp.float32),
    mesh=_mesh,
    scratch_types=[
        pltpu.VMEM((4, H, W), jnp.float32),
        pltpu.VMEM((4, H, W), jnp.float32),
        pltpu.VMEM((JOB, W), jnp.float32),
        pltpu.VMEM((JOB, W), jnp.float32),
        pltpu.SemaphoreType.DMA,
        pltpu.SemaphoreType.DMA,
        pltpu.SemaphoreType.DMA,
        pltpu.SemaphoreType.DMA,
    ],
)
def _sc_preds(preds_hbm, outp_hbm, pbuf0, pbuf1, obuf0, obuf1,
              sg0, sg1, so0, so1):
    w = lax.axis_index("s") * 2 + lax.axis_index("c")
    pbufs, obufs = (pbuf0, pbuf1), (obuf0, obuf1)
    sgs, sos = (sg0, sg1), (so0, so1)

    nstage = 6

    def stage_info(i):
        slot, st = i // 2, i % 2
        jid = w + 32 * slot
        return jid, st, jid < NJP

    def make_gathers(i):
        jid, st, valid = stage_info(i)
        pbuf, sg = pbufs[i % 2], sgs[i % 2]
        cps = []
        for k in range(4):
            u = 4 * jid + 2 * st + k // 2
            b = u // A
            a = u - b * A
            cps.append(pltpu.make_async_copy(
                preds_hbm.at[b, (k % 2) * A + a], pbuf.at[k], sg))
        return cps, valid

    def guarded(fn, valid):
        @pl.when(valid)
        def _():
            fn()

    def interleave(i):
        jid, st, valid = stage_info(i)
        pbuf = pbufs[i % 2]
        obuf = obufs[(i // 2) % 2]
        base = st * STG

        @pl.when(valid)
        def _():
            def body(h, _):
                for k in range(4):
                    row = base + (k // 2) * 74 + h * 2 + (k % 2)
                    for col in COLS:
                        obuf[row, pl.ds(col, 16)] = pbuf[k, h, pl.ds(col, 16)]
                return 0
            lax.fori_loop(0, H, body, 0)

    def make_out(i):
        jid, st, valid = stage_info(i)
        j = i // 2
        return pltpu.make_async_copy(
            obufs[j % 2], outp_hbm.at[pl.ds(jid * JOB, JOB)],
            sos[j % 2]), valid

    pending = {}
    g = make_gathers(0)
    guarded(lambda cps=g[0]: [c.start() for c in cps], g[1])
    for i in range(nstage):
        if i + 1 < nstage:
            gn = make_gathers(i + 1)
            guarded(lambda cps=gn[0]: [c.start() for c in cps], gn[1])
        else:
            gn = None
        guarded(lambda cps=g[0]: [c.wait() for c in cps], g[1])
        g = gn
        j = i // 2
        if i % 2 == 0 and (j - 2) in pending:
            cp, v = pending.pop(j - 2)
            guarded(lambda cp=cp: cp.wait(), v)
        interleave(i)
        if i % 2 == 1:
            cp, v = make_out(i)
            guarded(lambda cp=cp: cp.start(), v)
            pending[j] = (cp, v)
    for cp, v in pending.values():
        guarded(lambda cp=cp: cp.wait(), v)


OCT = B // 8
OROWS = RR // OCT


def _tc_regs_body(x_ref, o_ref):
    x = x_ref[...]
    y = x.reshape(4, A, H, 8, W).transpose(3, 1, 2, 0, 4)
    o_ref[...] = y.reshape(OROWS, W)


def _tc_regs(rt4):
    return pl.pallas_call(
        _tc_regs_body,
        out_shape=jax.ShapeDtypeStruct((RR, W), jnp.float32),
        grid=(OCT,),
        in_specs=[pl.BlockSpec((4 * A, H, 8, W), lambda q: (0, 0, q, 0))],
        out_specs=pl.BlockSpec((OROWS, W), lambda q: (q, 0)),
    )(rt4)


def kernel(preds, regs):
    bs, _, fh, fw = preds.shape
    rt4 = regs.transpose(1, 2, 0, 3)
    outr = _tc_regs(rt4)
    outp = _sc_preds(preds)
    return (
        outp.reshape(bs, A, fh, 2, fw).transpose(0, 1, 2, 4, 3),
        outr.reshape(bs, A, fh, 4, fw).transpose(0, 1, 2, 4, 3),
    )


# speedup vs baseline: 2.2701x; 1.0534x over previous
import functools

import jax
import jax.numpy as jnp
from jax import lax
from jax.experimental import pallas as pl
from jax.experimental.pallas import tpu as pltpu
from jax.experimental.pallas import tpu_sc as plsc

B, A, H, W = 32, 9, 37, 62
RP = B * A * H * 2
RR = B * A * H * 4
JOB = 296
STG = 148
NJP = RP // JOB
COLS = (0, 16, 32, 46)

_mesh = plsc.VectorSubcoreMesh(core_axis_name="c", subcore_axis_name="s")


@functools.partial(
    pl.kernel,
    out_type=jax.ShapeDtypeStruct((RP, W), jnp.float32),
    mesh=_mesh,
    scratch_types=[
        pltpu.VMEM((4, H, W), jnp.float32),
        pltpu.VMEM((4, H, W), jnp.float32),
        pltpu.VMEM((2 * JOB, W), jnp.float32),
        pltpu.SemaphoreType.DMA,
        pltpu.SemaphoreType.DMA,
        pltpu.SemaphoreType.DMA,
    ],
)
def _sc_preds(preds_hbm, outp_hbm, pbuf0, pbuf1, obuf, sg0, sg1, so):
    w = lax.axis_index("s") * 2 + lax.axis_index("c")
    pbufs, sgs = (pbuf0, pbuf1), (sg0, sg1)
    extra = w < NJP - 2 * 32

    def make_gathers(i, base_u):
        pbuf, sg = pbufs[i % 2], sgs[i % 2]
        cps = []
        for k in range(4):
            u = base_u + k // 2
            b = u // A
            a = u - b * A
            cps.append(pltpu.make_async_copy(
                preds_hbm.at[b, (k % 2) * A + a], pbuf.at[k], sg))
        return cps

    def interleave(i, obase):
        pbuf = pbufs[i % 2]

        def body(h, _):
            for k in range(4):
                row = obase + (k // 2) * 74 + h * 2 + (k % 2)
                for col in COLS:
                    obuf[row, pl.ds(col, 16)] = pbuf[k, h, pl.ds(col, 16)]
            return 0
        lax.fori_loop(0, H, body, 0)

    u0 = 8 * w
    g = make_gathers(0, u0 + 0)
    for c in g:
        c.start()
    for i in range(4):
        if i + 1 < 4:
            gn = make_gathers(i + 1, u0 + 2 * (i + 1))
            for c in gn:
                c.start()
        else:
            gn = None
        for c in g:
            c.wait()
        g = gn
        interleave(i, i * STG)
    big_out = pltpu.make_async_copy(
        obuf, outp_hbm.at[pl.ds(w * 2 * JOB, 2 * JOB)], so)
    big_out.start()

    ue = 4 * (64 + w)

    @pl.when(extra)
    def _():
        ge = make_gathers(0, ue)
        for c in ge:
            c.start()
    @pl.when(extra)
    def _():
        ge2 = make_gathers(1, ue + 2)
        for c in ge2:
            c.start()
    big_out.wait()

    @pl.when(extra)
    def _():
        g0 = make_gathers(0, ue)
        for c in g0:
            c.wait()
        interleave(0, 0)
        g1 = make_gathers(1, ue + 2)
        for c in g1:
            c.wait()
        interleave(1, STG)
        cp = pltpu.make_async_copy(
            obuf.at[pl.ds(0, JOB)],
            outp_hbm.at[pl.ds((64 + w) * JOB, JOB)], so)
        cp.start()
        cp.wait()


OCT = B // 8


def _tc_relayout_body(x_ref, o_ref):
    o_ref[...] = x_ref[...].transpose(2, 0, 1, 3)


def _tc_relayout(pt4):
    return pl.pallas_call(
        _tc_relayout_body,
        out_shape=jax.ShapeDtypeStruct((B, 2 * A, H, W), jnp.float32),
        grid=(OCT,),
        in_specs=[pl.BlockSpec((2 * A, H, 8, W), lambda q: (0, 0, q, 0))],
        out_specs=pl.BlockSpec((8, 2 * A, H, W), lambda q: (q, 0, 0, 0)),
    )(pt4)


def _tc_regs_body(x_ref, o_ref):
    x = x_ref[...]
    y = x.reshape(4, A, H, 8, W).transpose(3, 1, 2, 0, 4)
    o_ref[...] = y.reshape(RR // OCT, W)


def _tc_regs(rt4):
    return pl.pallas_call(
        _tc_regs_body,
        out_shape=jax.ShapeDtypeStruct((RR, W), jnp.float32),
        grid=(OCT,),
        in_specs=[pl.BlockSpec((4 * A, H, 8, W), lambda q: (0, 0, q, 0))],
        out_specs=pl.BlockSpec((RR // OCT, W), lambda q: (q, 0)),
    )(rt4)


def kernel(preds, regs):
    bs, _, fh, fw = preds.shape
    pt4 = preds.transpose(1, 2, 0, 3)
    rt4 = regs.transpose(1, 2, 0, 3)
    preds_bm = _tc_relayout(pt4)
    outp = _sc_preds(preds_bm)
    outr = _tc_regs(rt4)
    return (
        outp.reshape(bs, A, fh, 2, fw).transpose(0, 1, 2, 4, 3),
        outr.reshape(bs, A, fh, 4, fw).transpose(0, 1, 2, 4, 3),
    )
